# baseline (speedup 1.0000x reference)
import functools

import jax
import jax.numpy as jnp
from jax import lax
from jax.experimental import pallas as pl
from jax.experimental.pallas import tpu as pltpu
from jax.experimental.pallas import tpu_sc as plsc

N = 10000
E = 320000
FEATS = 128
HIDDEN = 64
CLASSES = 64
DEPTH = 10
ALPHA = 0.9
EMB_DIM = 7
HID2 = 11

CHUNK = 10000
NCHUNK = E // CHUNK
NPAIR = NCHUNK // 2
EGRP = CHUNK // 16
NGRP = N // 16
MASK14 = (1 << 14) - 1
BR = 10000


def _pack_body(edges_ref, packed_ref):
    e = edges_ref[...]
    packed_ref[...] = e[0:1, :] + (e[1:2, :] << 14)


def _in_proj_body(x_ref, W_ref, b_ref, out_ref):
    yT = lax.dot_general(W_ref[...], x_ref[...], (((0,), (1,)), ((), ())),
                         preferred_element_type=jnp.float32)
    out_ref[...] = yT + b_ref[...]


def _out_proj_body(xT_ref, W_ref, b_ref, out_ref):
    y = lax.dot_general(xT_ref[...], W_ref[...], (((0,), (0,)), ((), ())),
                        preferred_element_type=jnp.float32)
    out_ref[...] = y + b_ref[...]


def _mlp_body(d_ref, h_ref, emb_ref, W1_ref, b1_ref, W2_ref, b2_ref,
              out_ref):
    d = d_ref[...]
    h = h_ref[...]
    e = jnp.broadcast_to(emb_ref[0:1, :], (BR, EMB_DIM))
    z = jnp.concatenate([d, h, e], axis=1)
    z = z @ W1_ref[...] + b1_ref[...]
    z = jnp.where(z >= 0, z, 0.01 * z)
    z = (z @ W2_ref[...] + b2_ref[...]) / 2.0
    out_ref[...] = z


def _make_sc_body(phase_a):
    def _sc_body(xT_hbm, packed_hbm, dinv_io_hbm, out_hbm,
                 x0, x1, h0a, h0b, acc0, acc1, dinv, eb0, eb1,
                 sem0, sem1):
        wid = lax.axis_index("s") * 2 + lax.axis_index("c")
        c0 = wid * 2

        pltpu.sync_copy(xT_hbm.at[c0], x0)
        pltpu.sync_copy(xT_hbm.at[c0 + 1], x1)
        pltpu.sync_copy(xT_hbm.at[c0], h0a)
        pltpu.sync_copy(xT_hbm.at[c0 + 1], h0b)

        def stream_edges(group_fn):
            pltpu.async_copy(packed_hbm.at[pl.ds(0, CHUNK)], eb0, sem0)
            pltpu.async_copy(packed_hbm.at[pl.ds(CHUNK, CHUNK)], eb1, sem1)

            def pair_body(p, carry):
                base = p * (2 * CHUNK)
                pltpu.make_async_copy(
                    packed_hbm.at[pl.ds(base, CHUNK)], eb0, sem0).wait()

                @plsc.parallel_loop(0, EGRP, unroll=4)
                def _(g):
                    group_fn(eb0[pl.ds(g * 16, 16)])

                @pl.when(p + 1 < NPAIR)
                def _():
                    pltpu.async_copy(
                        packed_hbm.at[pl.ds(base + 2 * CHUNK, CHUNK)],
                        eb0, sem0)

                pltpu.make_async_copy(
                    packed_hbm.at[pl.ds(base + CHUNK, CHUNK)],
                    eb1, sem1).wait()

                @plsc.parallel_loop(0, EGRP, unroll=4)
                def _(g):
                    group_fn(eb1[pl.ds(g * 16, 16)])

                @pl.when(p + 1 < NPAIR)
                def _():
                    pltpu.async_copy(
                        packed_hbm.at[pl.ds(base + 3 * CHUNK, CHUNK)],
                        eb1, sem1)

                return carry

            lax.fori_loop(0, NPAIR, pair_body, 0)

        if phase_a:
            ones = jnp.full((16,), 1.0, jnp.float32)

            @plsc.parallel_loop(0, NGRP)
            def _(i):
                dinv[pl.ds(i * 16, 16)] = ones

            def deg_fn(pk):
                dstv = lax.shift_right_logical(pk, 14)
                plsc.addupdate_scatter(dinv, [dstv], ones)

            stream_edges(deg_fn)

            @plsc.parallel_loop(0, NGRP)
            def _(i):
                sl = pl.ds(i * 16, 16)
                v = dinv[sl]
                bits = lax.bitcast_convert_type(v, jnp.int32)
                y = lax.bitcast_convert_type(
                    jnp.int32(0x5F3759DF) - lax.shift_right_logical(bits, 1),
                    jnp.float32)
                for _unused in range(3):
                    y = y * (1.5 - 0.5 * v * y * y)
                dinv[sl] = y

            @pl.when(wid == 0)
            def _():
                pltpu.sync_copy(dinv, dinv_io_hbm)
        else:
            pltpu.sync_copy(dinv_io_hbm, dinv)

        def conv_fn(pk):
            srcv = pk & MASK14
            dstv = lax.shift_right_logical(pk, 14)
            nrm = (plsc.load_gather(dinv, [srcv])
                   * plsc.load_gather(dinv, [dstv]))
            plsc.addupdate_scatter(acc0, [dstv],
                                   plsc.load_gather(x0, [srcv]) * nrm)
            plsc.addupdate_scatter(acc1, [dstv],
                                   plsc.load_gather(x1, [srcv]) * nrm)

        def it_body(it, carry):
            @plsc.parallel_loop(0, NGRP)
            def _(i):
                sl = pl.ds(i * 16, 16)
                d2 = dinv[sl]
                d2 = d2 * d2
                acc0[sl] = d2 * x0[sl]
                acc1[sl] = d2 * x1[sl]

            stream_edges(conv_fn)

            @plsc.parallel_loop(0, NGRP)
            def _(i):
                sl = pl.ds(i * 16, 16)
                x0[sl] = ALPHA * acc0[sl] + (1.0 - ALPHA) * h0a[sl]
                x1[sl] = ALPHA * acc1[sl] + (1.0 - ALPHA) * h0b[sl]

            return carry

        lax.fori_loop(0, DEPTH, it_body, 0)

        pltpu.sync_copy(x0, out_hbm.at[c0])
        pltpu.sync_copy(x1, out_hbm.at[c0 + 1])

    return _sc_body


_SC_SCRATCH = [
    pltpu.VMEM((N,), jnp.float32),
    pltpu.VMEM((N,), jnp.float32),
    pltpu.VMEM((N,), jnp.float32),
    pltpu.VMEM((N,), jnp.float32),
    pltpu.VMEM((N,), jnp.float32),
    pltpu.VMEM((N,), jnp.float32),
    pltpu.VMEM((N,), jnp.float32),
    pltpu.VMEM((CHUNK,), jnp.int32),
    pltpu.VMEM((CHUNK,), jnp.int32),
    pltpu.SemaphoreType.DMA,
    pltpu.SemaphoreType.DMA,
]


def kernel(x, edges, Wdr, bdr, emb_table, W1, b1, W2, b2, Wtc, btc):
    packed = pl.pallas_call(
        _pack_body,
        out_shape=jax.ShapeDtypeStruct((1, E), jnp.int32),
    )(edges).reshape(E)

    xT = pl.pallas_call(
        _in_proj_body,
        out_shape=jax.ShapeDtypeStruct((HIDDEN, N), jnp.float32),
    )(x, Wdr, bdr.reshape(HIDDEN, 1))

    mesh = plsc.VectorSubcoreMesh(core_axis_name="c", subcore_axis_name="s")
    cp = pltpu.CompilerParams(needs_layout_passes=False)
    sc_a = pl.kernel(
        _make_sc_body(True),
        out_type=[
            jax.ShapeDtypeStruct((N,), jnp.float32),
            jax.ShapeDtypeStruct((HIDDEN, N), jnp.float32),
        ],
        mesh=mesh, compiler_params=cp, scratch_types=_SC_SCRATCH,
    )
    dinv_arr, x10T = sc_a(xT, packed)

    dcol = x10T.reshape(HIDDEN * N, 1)
    hcol = xT.reshape(HIDDEN * N, 1)
    xmT = pl.pallas_call(
        _mlp_body,
        grid=(HIDDEN * N // BR,),
        in_specs=[
            pl.BlockSpec((BR, 1), lambda i: (i, 0)),
            pl.BlockSpec((BR, 1), lambda i: (i, 0)),
            pl.BlockSpec((HIDDEN, EMB_DIM), lambda i: (0, 0)),
            pl.BlockSpec((2 + EMB_DIM, HID2), lambda i: (0, 0)),
            pl.BlockSpec((1, HID2), lambda i: (0, 0)),
            pl.BlockSpec((HID2, 1), lambda i: (0, 0)),
            pl.BlockSpec((1, 1), lambda i: (0, 0)),
        ],
        out_specs=pl.BlockSpec((BR, 1), lambda i: (i, 0)),
        out_shape=jax.ShapeDtypeStruct((HIDDEN * N, 1), jnp.float32),
    )(dcol, hcol, emb_table, W1, b1.reshape(1, HID2), W2,
      b2.reshape(1, 1)).reshape(HIDDEN, N)

    sc_b = pl.kernel(
        _make_sc_body(False),
        out_type=jax.ShapeDtypeStruct((HIDDEN, N), jnp.float32),
        mesh=mesh, compiler_params=cp, scratch_types=_SC_SCRATCH,
    )
    xT2 = sc_b(xmT, packed, dinv_arr)

    out = pl.pallas_call(
        _out_proj_body,
        out_shape=jax.ShapeDtypeStruct((N, CLASSES), jnp.float32),
    )(xT2, Wtc, btc.reshape(1, CLASSES))
    return out


# speedup vs baseline: 1.0772x; 1.0772x over previous
import functools

import jax
import jax.numpy as jnp
from jax import lax
from jax.experimental import pallas as pl
from jax.experimental.pallas import tpu as pltpu
from jax.experimental.pallas import tpu_sc as plsc

N = 10000
E = 320000
FEATS = 128
HIDDEN = 64
CLASSES = 64
DEPTH = 10
ALPHA = 0.9
EMB_DIM = 7
HID2 = 11

CHUNK = 10000
NCHUNK = E // CHUNK
NPAIR = NCHUNK // 2
EGRP = CHUNK // 16
NGRP = N // 16
MASK14 = (1 << 14) - 1
BR = 10000


def _pack_body(edges_ref, packed_ref):
    e = edges_ref[...]
    packed_ref[...] = e[0:1, :] + (e[1:2, :] << 14)


def _in_proj_body(x_ref, W_ref, b_ref, out_ref):
    yT = lax.dot_general(W_ref[...], x_ref[...], (((0,), (1,)), ((), ())),
                         preferred_element_type=jnp.float32)
    out_ref[...] = yT + b_ref[...]


def _out_proj_body(xT_ref, W_ref, b_ref, out_ref):
    y = lax.dot_general(xT_ref[...], W_ref[...], (((0,), (0,)), ((), ())),
                        preferred_element_type=jnp.float32)
    out_ref[...] = y + b_ref[...]


def _mlp_body(d_ref, h_ref, emb_ref, W1_ref, b1_ref, W2_ref, b2_ref,
              out_ref):
    d = d_ref[...]
    h = h_ref[...]
    e = jnp.broadcast_to(emb_ref[0:1, :], (BR, EMB_DIM))
    z = jnp.concatenate([d, h, e], axis=1)
    z = z @ W1_ref[...] + b1_ref[...]
    z = jnp.where(z >= 0, z, 0.01 * z)
    z = (z @ W2_ref[...] + b2_ref[...]) / 2.0
    out_ref[...] = z


def _make_sc_body(phase_a):
    def _sc_body(xT_hbm, packed_hbm, dinv_io_hbm, norm_io_hbm, out_hbm,
                 x0, x1, h0a, h0b, acc0, acc1, dinv, eb0, eb1, nb0, nb1,
                 sem0, sem1, sem2, sem3):
        wid = lax.axis_index("s") * 2 + lax.axis_index("c")
        c0 = wid * 2

        pltpu.sync_copy(xT_hbm.at[c0], x0)
        pltpu.sync_copy(xT_hbm.at[c0 + 1], x1)
        pltpu.sync_copy(xT_hbm.at[c0], h0a)
        pltpu.sync_copy(xT_hbm.at[c0 + 1], h0b)

        def stream_edges(group_fn, with_norm):
            pltpu.async_copy(packed_hbm.at[pl.ds(0, CHUNK)], eb0, sem0)
            pltpu.async_copy(packed_hbm.at[pl.ds(CHUNK, CHUNK)], eb1, sem1)
            if with_norm:
                pltpu.async_copy(norm_io_hbm.at[pl.ds(0, CHUNK)], nb0, sem2)
                pltpu.async_copy(norm_io_hbm.at[pl.ds(CHUNK, CHUNK)], nb1, sem3)

            def pair_body(p, carry):
                base = p * (2 * CHUNK)
                pltpu.make_async_copy(
                    packed_hbm.at[pl.ds(base, CHUNK)], eb0, sem0).wait()
                if with_norm:
                    pltpu.make_async_copy(
                        norm_io_hbm.at[pl.ds(base, CHUNK)], nb0, sem2).wait()

                    @plsc.parallel_loop(0, EGRP)
                    def _(g):
                        group_fn(eb0[pl.ds(g * 16, 16)],
                                 nb0[pl.ds(g * 16, 16)])
                else:
                    @plsc.parallel_loop(0, EGRP)
                    def _(g):
                        group_fn(eb0[pl.ds(g * 16, 16)], None)

                @pl.when(p + 1 < NPAIR)
                def _():
                    pltpu.async_copy(
                        packed_hbm.at[pl.ds(base + 2 * CHUNK, CHUNK)],
                        eb0, sem0)
                    if with_norm:
                        pltpu.async_copy(
                            norm_io_hbm.at[pl.ds(base + 2 * CHUNK, CHUNK)],
                            nb0, sem2)

                pltpu.make_async_copy(
                    packed_hbm.at[pl.ds(base + CHUNK, CHUNK)],
                    eb1, sem1).wait()
                if with_norm:
                    pltpu.make_async_copy(
                        norm_io_hbm.at[pl.ds(base + CHUNK, CHUNK)],
                        nb1, sem3).wait()

                    @plsc.parallel_loop(0, EGRP)
                    def _(g):
                        group_fn(eb1[pl.ds(g * 16, 16)],
                                 nb1[pl.ds(g * 16, 16)])
                else:
                    @plsc.parallel_loop(0, EGRP)
                    def _(g):
                        group_fn(eb1[pl.ds(g * 16, 16)], None)

                @pl.when(p + 1 < NPAIR)
                def _():
                    pltpu.async_copy(
                        packed_hbm.at[pl.ds(base + 3 * CHUNK, CHUNK)],
                        eb1, sem1)
                    if with_norm:
                        pltpu.async_copy(
                            norm_io_hbm.at[pl.ds(base + 3 * CHUNK, CHUNK)],
                            nb1, sem3)

                return carry

            lax.fori_loop(0, NPAIR, pair_body, 0)

        if phase_a:
            ones = jnp.full((16,), 1.0, jnp.float32)

            @plsc.parallel_loop(0, NGRP)
            def _(i):
                dinv[pl.ds(i * 16, 16)] = ones

            def deg_fn(pk, _nr):
                dstv = lax.shift_right_logical(pk, 14)
                plsc.addupdate_scatter(dinv, [dstv], ones)

            stream_edges(deg_fn, False)

            @plsc.parallel_loop(0, NGRP)
            def _(i):
                sl = pl.ds(i * 16, 16)
                v = dinv[sl]
                bits = lax.bitcast_convert_type(v, jnp.int32)
                y = lax.bitcast_convert_type(
                    jnp.int32(0x5F3759DF) - lax.shift_right_logical(bits, 1),
                    jnp.float32)
                for _unused in range(3):
                    y = y * (1.5 - 0.5 * v * y * y)
                dinv[sl] = y

            @pl.when(wid == 0)
            def _():
                pltpu.sync_copy(dinv, dinv_io_hbm)

            def nchunk_body(ch, carry):
                pltpu.sync_copy(packed_hbm.at[pl.ds(ch * CHUNK, CHUNK)], eb0)

                @plsc.parallel_loop(0, EGRP)
                def _(g):
                    sl = pl.ds(g * 16, 16)
                    pk = eb0[sl]
                    srcv = pk & MASK14
                    dstv = lax.shift_right_logical(pk, 14)
                    nb0[sl] = (plsc.load_gather(dinv, [srcv])
                               * plsc.load_gather(dinv, [dstv]))

                pltpu.sync_copy(nb0, norm_io_hbm.at[pl.ds(ch * CHUNK, CHUNK)])
                return carry

            lax.fori_loop(0, NCHUNK, nchunk_body, 0)
        else:
            pltpu.sync_copy(dinv_io_hbm, dinv)

        def conv_fn(pk, nrm):
            srcv = pk & MASK14
            dstv = lax.shift_right_logical(pk, 14)
            plsc.addupdate_scatter(acc0, [dstv],
                                   plsc.load_gather(x0, [srcv]) * nrm)
            plsc.addupdate_scatter(acc1, [dstv],
                                   plsc.load_gather(x1, [srcv]) * nrm)

        def it_body(it, carry):
            @plsc.parallel_loop(0, NGRP)
            def _(i):
                sl = pl.ds(i * 16, 16)
                d2 = dinv[sl]
                d2 = d2 * d2
                acc0[sl] = d2 * x0[sl]
                acc1[sl] = d2 * x1[sl]

            stream_edges(conv_fn, True)

            @plsc.parallel_loop(0, NGRP)
            def _(i):
                sl = pl.ds(i * 16, 16)
                x0[sl] = ALPHA * acc0[sl] + (1.0 - ALPHA) * h0a[sl]
                x1[sl] = ALPHA * acc1[sl] + (1.0 - ALPHA) * h0b[sl]

            return carry

        lax.fori_loop(0, DEPTH, it_body, 0)

        pltpu.sync_copy(x0, out_hbm.at[c0])
        pltpu.sync_copy(x1, out_hbm.at[c0 + 1])

    return _sc_body


_SC_SCRATCH = [
    pltpu.VMEM((N,), jnp.float32),
    pltpu.VMEM((N,), jnp.float32),
    pltpu.VMEM((N,), jnp.float32),
    pltpu.VMEM((N,), jnp.float32),
    pltpu.VMEM((N,), jnp.float32),
    pltpu.VMEM((N,), jnp.float32),
    pltpu.VMEM((N,), jnp.float32),
    pltpu.VMEM((CHUNK,), jnp.int32),
    pltpu.VMEM((CHUNK,), jnp.int32),
    pltpu.VMEM((CHUNK,), jnp.float32),
    pltpu.VMEM((CHUNK,), jnp.float32),
    pltpu.SemaphoreType.DMA,
    pltpu.SemaphoreType.DMA,
    pltpu.SemaphoreType.DMA,
    pltpu.SemaphoreType.DMA,
]


def kernel(x, edges, Wdr, bdr, emb_table, W1, b1, W2, b2, Wtc, btc):
    packed = pl.pallas_call(
        _pack_body,
        out_shape=jax.ShapeDtypeStruct((1, E), jnp.int32),
    )(edges).reshape(E)

    xT = pl.pallas_call(
        _in_proj_body,
        out_shape=jax.ShapeDtypeStruct((HIDDEN, N), jnp.float32),
    )(x, Wdr, bdr.reshape(HIDDEN, 1))

    mesh = plsc.VectorSubcoreMesh(core_axis_name="c", subcore_axis_name="s")
    cp = pltpu.CompilerParams(needs_layout_passes=False)
    sc_a = pl.kernel(
        _make_sc_body(True),
        out_type=[
            jax.ShapeDtypeStruct((N,), jnp.float32),
            jax.ShapeDtypeStruct((E,), jnp.float32),
            jax.ShapeDtypeStruct((HIDDEN, N), jnp.float32),
        ],
        mesh=mesh, compiler_params=cp, scratch_types=_SC_SCRATCH,
    )
    dinv_arr, norm_arr, x10T = sc_a(xT, packed)

    dcol = x10T.reshape(HIDDEN * N, 1)
    hcol = xT.reshape(HIDDEN * N, 1)
    xmT = pl.pallas_call(
        _mlp_body,
        grid=(HIDDEN * N // BR,),
        in_specs=[
            pl.BlockSpec((BR, 1), lambda i: (i, 0)),
            pl.BlockSpec((BR, 1), lambda i: (i, 0)),
            pl.BlockSpec((HIDDEN, EMB_DIM), lambda i: (0, 0)),
            pl.BlockSpec((2 + EMB_DIM, HID2), lambda i: (0, 0)),
            pl.BlockSpec((1, HID2), lambda i: (0, 0)),
            pl.BlockSpec((HID2, 1), lambda i: (0, 0)),
            pl.BlockSpec((1, 1), lambda i: (0, 0)),
        ],
        out_specs=pl.BlockSpec((BR, 1), lambda i: (i, 0)),
        out_shape=jax.ShapeDtypeStruct((HIDDEN * N, 1), jnp.float32),
    )(dcol, hcol, emb_table, W1, b1.reshape(1, HID2), W2,
      b2.reshape(1, 1)).reshape(HIDDEN, N)

    sc_b = pl.kernel(
        _make_sc_body(False),
        out_type=jax.ShapeDtypeStruct((HIDDEN, N), jnp.float32),
        mesh=mesh, compiler_params=cp, scratch_types=_SC_SCRATCH,
    )
    xT2 = sc_b(xmT, packed, dinv_arr, norm_arr)

    out = pl.pallas_call(
        _out_proj_body,
        out_shape=jax.ShapeDtypeStruct((N, CLASSES), jnp.float32),
    )(xT2, Wtc, btc.reshape(1, CLASSES))
    return out


# speedup vs baseline: 1.0941x; 1.0157x over previous
import functools

import jax
import jax.numpy as jnp
from jax import lax
from jax.experimental import pallas as pl
from jax.experimental.pallas import tpu as pltpu
from jax.experimental.pallas import tpu_sc as plsc

N = 10000
E = 320000
FEATS = 128
HIDDEN = 64
CLASSES = 64
DEPTH = 10
ALPHA = 0.9
EMB_DIM = 7
HID2 = 11

CHUNK = 10000
NCHUNK = E // CHUNK
NPAIR = NCHUNK // 2
EGRP = CHUNK // 16
NGRP = N // 16
MASK14 = (1 << 14) - 1
BR = 10000


def _pack_body(edges_ref, packed_ref):
    e = edges_ref[...]
    packed_ref[...] = e[0:1, :] + (e[1:2, :] << 14)


def _in_proj_body(x_ref, W_ref, b_ref, out_ref):
    yT = lax.dot_general(W_ref[...], x_ref[...], (((0,), (1,)), ((), ())),
                         preferred_element_type=jnp.float32)
    out_ref[...] = yT + b_ref[...]


def _out_proj_body(xT_ref, W_ref, b_ref, out_ref):
    y = lax.dot_general(xT_ref[...], W_ref[...], (((0,), (0,)), ((), ())),
                        preferred_element_type=jnp.float32)
    out_ref[...] = y + b_ref[...]


def _mlp_body(d_ref, h_ref, emb_ref, W1_ref, b1_ref, W2_ref, b2_ref,
              out_ref):
    d = d_ref[...]
    h = h_ref[...]
    e = jnp.broadcast_to(emb_ref[0:1, :], (BR, EMB_DIM))
    z = jnp.concatenate([d, h, e], axis=1)
    z = z @ W1_ref[...] + b1_ref[...]
    z = jnp.where(z >= 0, z, 0.01 * z)
    z = (z @ W2_ref[...] + b2_ref[...]) / 2.0
    out_ref[...] = z


def _make_sc_body(phase_a):
    def _sc_body(xT_hbm, packed_hbm, dinv_io_hbm, norm_io_hbm, out_hbm,
                 x0, x1, h0a, h0b, acc0, acc1, dinv, eb0, eb1, nb0, nb1,
                 sem0, sem1, sem2, sem3):
        wid = lax.axis_index("s") * 2 + lax.axis_index("c")
        c0 = wid * 2

        pltpu.sync_copy(xT_hbm.at[c0], x0)
        pltpu.sync_copy(xT_hbm.at[c0 + 1], x1)
        pltpu.sync_copy(xT_hbm.at[c0], h0a)
        pltpu.sync_copy(xT_hbm.at[c0 + 1], h0b)

        def stream_edges(group_fn, with_norm):
            pltpu.async_copy(packed_hbm.at[pl.ds(0, CHUNK)], eb0, sem0)
            pltpu.async_copy(packed_hbm.at[pl.ds(CHUNK, CHUNK)], eb1, sem1)
            if with_norm:
                pltpu.async_copy(norm_io_hbm.at[pl.ds(0, CHUNK)], nb0, sem2)
                pltpu.async_copy(norm_io_hbm.at[pl.ds(CHUNK, CHUNK)], nb1, sem3)

            def pair_body(p, carry):
                base = p * (2 * CHUNK)
                pltpu.make_async_copy(
                    packed_hbm.at[pl.ds(base, CHUNK)], eb0, sem0).wait()
                if with_norm:
                    pltpu.make_async_copy(
                        norm_io_hbm.at[pl.ds(base, CHUNK)], nb0, sem2).wait()

                    @plsc.parallel_loop(0, EGRP)
                    def _(g):
                        group_fn(eb0[pl.ds(g * 16, 16)],
                                 nb0[pl.ds(g * 16, 16)])
                else:
                    @plsc.parallel_loop(0, EGRP)
                    def _(g):
                        group_fn(eb0[pl.ds(g * 16, 16)], None)

                @pl.when(p + 1 < NPAIR)
                def _():
                    pltpu.async_copy(
                        packed_hbm.at[pl.ds(base + 2 * CHUNK, CHUNK)],
                        eb0, sem0)
                    if with_norm:
                        pltpu.async_copy(
                            norm_io_hbm.at[pl.ds(base + 2 * CHUNK, CHUNK)],
                            nb0, sem2)

                pltpu.make_async_copy(
                    packed_hbm.at[pl.ds(base + CHUNK, CHUNK)],
                    eb1, sem1).wait()
                if with_norm:
                    pltpu.make_async_copy(
                        norm_io_hbm.at[pl.ds(base + CHUNK, CHUNK)],
                        nb1, sem3).wait()

                    @plsc.parallel_loop(0, EGRP)
                    def _(g):
                        group_fn(eb1[pl.ds(g * 16, 16)],
                                 nb1[pl.ds(g * 16, 16)])
                else:
                    @plsc.parallel_loop(0, EGRP)
                    def _(g):
                        group_fn(eb1[pl.ds(g * 16, 16)], None)

                @pl.when(p + 1 < NPAIR)
                def _():
                    pltpu.async_copy(
                        packed_hbm.at[pl.ds(base + 3 * CHUNK, CHUNK)],
                        eb1, sem1)
                    if with_norm:
                        pltpu.async_copy(
                            norm_io_hbm.at[pl.ds(base + 3 * CHUNK, CHUNK)],
                            nb1, sem3)

                return carry

            lax.fori_loop(0, NPAIR, pair_body, 0)

        if phase_a:
            ones = jnp.full((16,), 1.0, jnp.float32)

            @plsc.parallel_loop(0, NGRP)
            def _(i):
                dinv[pl.ds(i * 16, 16)] = ones

            def deg_fn(pk, _nr):
                dstv = lax.shift_right_logical(pk, 14)
                plsc.addupdate_scatter(dinv, [dstv], ones)

            stream_edges(deg_fn, False)

            @plsc.parallel_loop(0, NGRP)
            def _(i):
                sl = pl.ds(i * 16, 16)
                v = dinv[sl]
                bits = lax.bitcast_convert_type(v, jnp.int32)
                y = lax.bitcast_convert_type(
                    jnp.int32(0x5F3759DF) - lax.shift_right_logical(bits, 1),
                    jnp.float32)
                for _unused in range(3):
                    y = y * (1.5 - 0.5 * v * y * y)
                dinv[sl] = y

            @pl.when(wid == 0)
            def _():
                pltpu.sync_copy(dinv, dinv_io_hbm)

            pltpu.async_copy(packed_hbm.at[pl.ds(0, CHUNK)], eb0, sem0)
            pltpu.async_copy(packed_hbm.at[pl.ds(CHUNK, CHUNK)], eb1, sem1)

            def _norm_groups(eb, nb):
                @plsc.parallel_loop(0, EGRP)
                def _(g):
                    sl = pl.ds(g * 16, 16)
                    pk = eb[sl]
                    srcv = pk & MASK14
                    dstv = lax.shift_right_logical(pk, 14)
                    nb[sl] = (plsc.load_gather(dinv, [srcv])
                              * plsc.load_gather(dinv, [dstv]))

            def npair_body(pr, carry):
                base = pr * (2 * CHUNK)
                pltpu.make_async_copy(
                    packed_hbm.at[pl.ds(base, CHUNK)], eb0, sem0).wait()
                @pl.when(pr > 0)
                def _():
                    pltpu.make_async_copy(
                        nb0, norm_io_hbm.at[pl.ds(base - 2 * CHUNK, CHUNK)],
                        sem2).wait()
                _norm_groups(eb0, nb0)
                pltpu.async_copy(
                    nb0, norm_io_hbm.at[pl.ds(base, CHUNK)], sem2)
                @pl.when(pr + 1 < NPAIR)
                def _():
                    pltpu.async_copy(
                        packed_hbm.at[pl.ds(base + 2 * CHUNK, CHUNK)],
                        eb0, sem0)

                pltpu.make_async_copy(
                    packed_hbm.at[pl.ds(base + CHUNK, CHUNK)], eb1, sem1).wait()
                @pl.when(pr > 0)
                def _():
                    pltpu.make_async_copy(
                        nb1, norm_io_hbm.at[pl.ds(base - CHUNK, CHUNK)],
                        sem3).wait()
                _norm_groups(eb1, nb1)
                pltpu.async_copy(
                    nb1, norm_io_hbm.at[pl.ds(base + CHUNK, CHUNK)], sem3)
                @pl.when(pr + 1 < NPAIR)
                def _():
                    pltpu.async_copy(
                        packed_hbm.at[pl.ds(base + 3 * CHUNK, CHUNK)],
                        eb1, sem1)
                return carry

            lax.fori_loop(0, NPAIR, npair_body, 0)
            pltpu.make_async_copy(
                nb0, norm_io_hbm.at[pl.ds((NCHUNK - 2) * CHUNK, CHUNK)],
                sem2).wait()
            pltpu.make_async_copy(
                nb1, norm_io_hbm.at[pl.ds((NCHUNK - 1) * CHUNK, CHUNK)],
                sem3).wait()
        else:
            pltpu.sync_copy(dinv_io_hbm, dinv)

        def conv_fn(pk, nrm):
            srcv = pk & MASK14
            dstv = lax.shift_right_logical(pk, 14)
            plsc.addupdate_scatter(acc0, [dstv],
                                   plsc.load_gather(x0, [srcv]) * nrm)
            plsc.addupdate_scatter(acc1, [dstv],
                                   plsc.load_gather(x1, [srcv]) * nrm)

        def it_body(it, carry):
            @plsc.parallel_loop(0, NGRP)
            def _(i):
                sl = pl.ds(i * 16, 16)
                d2 = dinv[sl]
                d2 = d2 * d2
                acc0[sl] = d2 * x0[sl]
                acc1[sl] = d2 * x1[sl]

            stream_edges(conv_fn, True)

            @plsc.parallel_loop(0, NGRP)
            def _(i):
                sl = pl.ds(i * 16, 16)
                x0[sl] = ALPHA * acc0[sl] + (1.0 - ALPHA) * h0a[sl]
                x1[sl] = ALPHA * acc1[sl] + (1.0 - ALPHA) * h0b[sl]

            return carry

        lax.fori_loop(0, DEPTH, it_body, 0)

        pltpu.sync_copy(x0, out_hbm.at[c0])
        pltpu.sync_copy(x1, out_hbm.at[c0 + 1])

    return _sc_body


_SC_SCRATCH = [
    pltpu.VMEM((N,), jnp.float32),
    pltpu.VMEM((N,), jnp.float32),
    pltpu.VMEM((N,), jnp.float32),
    pltpu.VMEM((N,), jnp.float32),
    pltpu.VMEM((N,), jnp.float32),
    pltpu.VMEM((N,), jnp.float32),
    pltpu.VMEM((N,), jnp.float32),
    pltpu.VMEM((CHUNK,), jnp.int32),
    pltpu.VMEM((CHUNK,), jnp.int32),
    pltpu.VMEM((CHUNK,), jnp.float32),
    pltpu.VMEM((CHUNK,), jnp.float32),
    pltpu.SemaphoreType.DMA,
    pltpu.SemaphoreType.DMA,
    pltpu.SemaphoreType.DMA,
    pltpu.SemaphoreType.DMA,
]


def kernel(x, edges, Wdr, bdr, emb_table, W1, b1, W2, b2, Wtc, btc):
    packed = pl.pallas_call(
        _pack_body,
        out_shape=jax.ShapeDtypeStruct((1, E), jnp.int32),
    )(edges).reshape(E)

    xT = pl.pallas_call(
        _in_proj_body,
        out_shape=jax.ShapeDtypeStruct((HIDDEN, N), jnp.float32),
    )(x, Wdr, bdr.reshape(HIDDEN, 1))

    mesh = plsc.VectorSubcoreMesh(core_axis_name="c", subcore_axis_name="s")
    cp = pltpu.CompilerParams(needs_layout_passes=False)
    sc_a = pl.kernel(
        _make_sc_body(True),
        out_type=[
            jax.ShapeDtypeStruct((N,), jnp.float32),
            jax.ShapeDtypeStruct((E,), jnp.float32),
            jax.ShapeDtypeStruct((HIDDEN, N), jnp.float32),
        ],
        mesh=mesh, compiler_params=cp, scratch_types=_SC_SCRATCH,
    )
    dinv_arr, norm_arr, x10T = sc_a(xT, packed)

    dcol = x10T.reshape(HIDDEN * N, 1)
    hcol = xT.reshape(HIDDEN * N, 1)
    xmT = pl.pallas_call(
        _mlp_body,
        grid=(HIDDEN * N // BR,),
        in_specs=[
            pl.BlockSpec((BR, 1), lambda i: (i, 0)),
            pl.BlockSpec((BR, 1), lambda i: (i, 0)),
            pl.BlockSpec((HIDDEN, EMB_DIM), lambda i: (0, 0)),
            pl.BlockSpec((2 + EMB_DIM, HID2), lambda i: (0, 0)),
            pl.BlockSpec((1, HID2), lambda i: (0, 0)),
            pl.BlockSpec((HID2, 1), lambda i: (0, 0)),
            pl.BlockSpec((1, 1), lambda i: (0, 0)),
        ],
        out_specs=pl.BlockSpec((BR, 1), lambda i: (i, 0)),
        out_shape=jax.ShapeDtypeStruct((HIDDEN * N, 1), jnp.float32),
    )(dcol, hcol, emb_table, W1, b1.reshape(1, HID2), W2,
      b2.reshape(1, 1)).reshape(HIDDEN, N)

    sc_b = pl.kernel(
        _make_sc_body(False),
        out_type=jax.ShapeDtypeStruct((HIDDEN, N), jnp.float32),
        mesh=mesh, compiler_params=cp, scratch_types=_SC_SCRATCH,
    )
    xT2 = sc_b(xmT, packed, dinv_arr, norm_arr)

    out = pl.pallas_call(
        _out_proj_body,
        out_shape=jax.ShapeDtypeStruct((N, CLASSES), jnp.float32),
    )(xT2, Wtc, btc.reshape(1, CLASSES))
    return out


# speedup vs baseline: 1.1008x; 1.0061x over previous
import functools

import jax
import jax.numpy as jnp
from jax import lax
from jax.experimental import pallas as pl
from jax.experimental.pallas import tpu as pltpu
from jax.experimental.pallas import tpu_sc as plsc

N = 10000
E = 320000
FEATS = 128
HIDDEN = 64
CLASSES = 64
DEPTH = 10
ALPHA = 0.9
EMB_DIM = 7
HID2 = 11

CHUNK = 10000
NCHUNK = E // CHUNK
NPAIR = NCHUNK // 2
EGRP = CHUNK // 16
NGRP = N // 16
MASK14 = (1 << 14) - 1
BR = 10000


def _pack_body(edges_ref, packed_ref):
    e = edges_ref[...]
    packed_ref[...] = e[0:1, :] + (e[1:2, :] << 14)


def _in_proj_body(x_ref, W_ref, b_ref, out_ref):
    yT = lax.dot_general(W_ref[...], x_ref[...], (((0,), (1,)), ((), ())),
                         preferred_element_type=jnp.float32)
    out_ref[...] = yT + b_ref[...]


def _out_proj_body(xT_ref, W_ref, b_ref, out_ref):
    y = lax.dot_general(xT_ref[...], W_ref[...], (((0,), (0,)), ((), ())),
                        preferred_element_type=jnp.float32)
    out_ref[...] = y + b_ref[...]


def _mlp_body(d_ref, h_ref, emb_ref, W1_ref, b1_ref, W2_ref, b2_ref,
              out_ref):
    d = d_ref[...]
    h = h_ref[...]
    e = jnp.broadcast_to(emb_ref[0:1, :], (BR, EMB_DIM))
    z = jnp.concatenate([d, h, e], axis=1)
    z = z @ W1_ref[...] + b1_ref[...]
    z = jnp.where(z >= 0, z, 0.01 * z)
    z = (z @ W2_ref[...] + b2_ref[...]) / 2.0
    out_ref[...] = z


def _make_sc_body(phase_a):
    def _sc_body(xT_hbm, packed_hbm, dinv_io_hbm, norm_io_hbm, out_hbm,
                 x0, x1, h0a, h0b, acc0, acc1, dinv, eb0, eb1, nb0, nb1,
                 sem0, sem1, sem2, sem3):
        wid = lax.axis_index("s") * 2 + lax.axis_index("c")
        c0 = wid * 2

        pltpu.sync_copy(xT_hbm.at[c0], x0)
        pltpu.sync_copy(xT_hbm.at[c0 + 1], x1)
        pltpu.sync_copy(xT_hbm.at[c0], h0a)
        pltpu.sync_copy(xT_hbm.at[c0 + 1], h0b)

        def stream_edges(group_fn, with_norm):
            pltpu.async_copy(packed_hbm.at[pl.ds(0, CHUNK)], eb0, sem0)
            pltpu.async_copy(packed_hbm.at[pl.ds(CHUNK, CHUNK)], eb1, sem1)
            if with_norm:
                pltpu.async_copy(norm_io_hbm.at[pl.ds(0, CHUNK)], nb0, sem2)
                pltpu.async_copy(norm_io_hbm.at[pl.ds(CHUNK, CHUNK)], nb1, sem3)

            def pair_body(p, carry):
                base = p * (2 * CHUNK)
                pltpu.make_async_copy(
                    packed_hbm.at[pl.ds(base, CHUNK)], eb0, sem0).wait()
                if with_norm:
                    pltpu.make_async_copy(
                        norm_io_hbm.at[pl.ds(base, CHUNK)], nb0, sem2).wait()

                    @plsc.parallel_loop(0, EGRP, unroll=2)
                    def _(g):
                        group_fn(eb0[pl.ds(g * 16, 16)],
                                 nb0[pl.ds(g * 16, 16)])
                else:
                    @plsc.parallel_loop(0, EGRP, unroll=2)
                    def _(g):
                        group_fn(eb0[pl.ds(g * 16, 16)], None)

                @pl.when(p + 1 < NPAIR)
                def _():
                    pltpu.async_copy(
                        packed_hbm.at[pl.ds(base + 2 * CHUNK, CHUNK)],
                        eb0, sem0)
                    if with_norm:
                        pltpu.async_copy(
                            norm_io_hbm.at[pl.ds(base + 2 * CHUNK, CHUNK)],
                            nb0, sem2)

                pltpu.make_async_copy(
                    packed_hbm.at[pl.ds(base + CHUNK, CHUNK)],
                    eb1, sem1).wait()
                if with_norm:
                    pltpu.make_async_copy(
                        norm_io_hbm.at[pl.ds(base + CHUNK, CHUNK)],
                        nb1, sem3).wait()

                    @plsc.parallel_loop(0, EGRP, unroll=2)
                    def _(g):
                        group_fn(eb1[pl.ds(g * 16, 16)],
                                 nb1[pl.ds(g * 16, 16)])
                else:
                    @plsc.parallel_loop(0, EGRP, unroll=2)
                    def _(g):
                        group_fn(eb1[pl.ds(g * 16, 16)], None)

                @pl.when(p + 1 < NPAIR)
                def _():
                    pltpu.async_copy(
                        packed_hbm.at[pl.ds(base + 3 * CHUNK, CHUNK)],
                        eb1, sem1)
                    if with_norm:
                        pltpu.async_copy(
                            norm_io_hbm.at[pl.ds(base + 3 * CHUNK, CHUNK)],
                            nb1, sem3)

                return carry

            lax.fori_loop(0, NPAIR, pair_body, 0)

        if phase_a:
            ones = jnp.full((16,), 1.0, jnp.float32)

            @plsc.parallel_loop(0, NGRP)
            def _(i):
                dinv[pl.ds(i * 16, 16)] = ones

            def deg_fn(pk, _nr):
                dstv = lax.shift_right_logical(pk, 14)
                plsc.addupdate_scatter(dinv, [dstv], ones)

            stream_edges(deg_fn, False)

            @plsc.parallel_loop(0, NGRP)
            def _(i):
                sl = pl.ds(i * 16, 16)
                v = dinv[sl]
                bits = lax.bitcast_convert_type(v, jnp.int32)
                y = lax.bitcast_convert_type(
                    jnp.int32(0x5F3759DF) - lax.shift_right_logical(bits, 1),
                    jnp.float32)
                for _unused in range(3):
                    y = y * (1.5 - 0.5 * v * y * y)
                dinv[sl] = y

            @pl.when(wid == 0)
            def _():
                pltpu.sync_copy(dinv, dinv_io_hbm)

            pltpu.async_copy(packed_hbm.at[pl.ds(0, CHUNK)], eb0, sem0)
            pltpu.async_copy(packed_hbm.at[pl.ds(CHUNK, CHUNK)], eb1, sem1)

            def _norm_groups(eb, nb):
                @plsc.parallel_loop(0, EGRP, unroll=2)
                def _(g):
                    sl = pl.ds(g * 16, 16)
                    pk = eb[sl]
                    srcv = pk & MASK14
                    dstv = lax.shift_right_logical(pk, 14)
                    nb[sl] = (plsc.load_gather(dinv, [srcv])
                              * plsc.load_gather(dinv, [dstv]))

            def npair_body(pr, carry):
                base = pr * (2 * CHUNK)
                pltpu.make_async_copy(
                    packed_hbm.at[pl.ds(base, CHUNK)], eb0, sem0).wait()
                @pl.when(pr > 0)
                def _():
                    pltpu.make_async_copy(
                        nb0, norm_io_hbm.at[pl.ds(base - 2 * CHUNK, CHUNK)],
                        sem2).wait()
                _norm_groups(eb0, nb0)
                pltpu.async_copy(
                    nb0, norm_io_hbm.at[pl.ds(base, CHUNK)], sem2)
                @pl.when(pr + 1 < NPAIR)
                def _():
                    pltpu.async_copy(
                        packed_hbm.at[pl.ds(base + 2 * CHUNK, CHUNK)],
                        eb0, sem0)

                pltpu.make_async_copy(
                    packed_hbm.at[pl.ds(base + CHUNK, CHUNK)], eb1, sem1).wait()
                @pl.when(pr > 0)
                def _():
                    pltpu.make_async_copy(
                        nb1, norm_io_hbm.at[pl.ds(base - CHUNK, CHUNK)],
                        sem3).wait()
                _norm_groups(eb1, nb1)
                pltpu.async_copy(
                    nb1, norm_io_hbm.at[pl.ds(base + CHUNK, CHUNK)], sem3)
                @pl.when(pr + 1 < NPAIR)
                def _():
                    pltpu.async_copy(
                        packed_hbm.at[pl.ds(base + 3 * CHUNK, CHUNK)],
                        eb1, sem1)
                return carry

            lax.fori_loop(0, NPAIR, npair_body, 0)
            pltpu.make_async_copy(
                nb0, norm_io_hbm.at[pl.ds((NCHUNK - 2) * CHUNK, CHUNK)],
                sem2).wait()
            pltpu.make_async_copy(
                nb1, norm_io_hbm.at[pl.ds((NCHUNK - 1) * CHUNK, CHUNK)],
                sem3).wait()
        else:
            pltpu.sync_copy(dinv_io_hbm, dinv)

        def conv_fn(pk, nrm):
            srcv = pk & MASK14
            dstv = lax.shift_right_logical(pk, 14)
            plsc.addupdate_scatter(acc0, [dstv],
                                   plsc.load_gather(x0, [srcv]) * nrm)
            plsc.addupdate_scatter(acc1, [dstv],
                                   plsc.load_gather(x1, [srcv]) * nrm)

        def it_body(it, carry):
            @plsc.parallel_loop(0, NGRP)
            def _(i):
                sl = pl.ds(i * 16, 16)
                d2 = dinv[sl]
                d2 = d2 * d2
                acc0[sl] = d2 * x0[sl]
                acc1[sl] = d2 * x1[sl]

            stream_edges(conv_fn, True)

            @plsc.parallel_loop(0, NGRP)
            def _(i):
                sl = pl.ds(i * 16, 16)
                x0[sl] = ALPHA * acc0[sl] + (1.0 - ALPHA) * h0a[sl]
                x1[sl] = ALPHA * acc1[sl] + (1.0 - ALPHA) * h0b[sl]

            return carry

        lax.fori_loop(0, DEPTH, it_body, 0)

        pltpu.sync_copy(x0, out_hbm.at[c0])
        pltpu.sync_copy(x1, out_hbm.at[c0 + 1])

    return _sc_body


_SC_SCRATCH = [
    pltpu.VMEM((N,), jnp.float32),
    pltpu.VMEM((N,), jnp.float32),
    pltpu.VMEM((N,), jnp.float32),
    pltpu.VMEM((N,), jnp.float32),
    pltpu.VMEM((N,), jnp.float32),
    pltpu.VMEM((N,), jnp.float32),
    pltpu.VMEM((N,), jnp.float32),
    pltpu.VMEM((CHUNK,), jnp.int32),
    pltpu.VMEM((CHUNK,), jnp.int32),
    pltpu.VMEM((CHUNK,), jnp.float32),
    pltpu.VMEM((CHUNK,), jnp.float32),
    pltpu.SemaphoreType.DMA,
    pltpu.SemaphoreType.DMA,
    pltpu.SemaphoreType.DMA,
    pltpu.SemaphoreType.DMA,
]


def kernel(x, edges, Wdr, bdr, emb_table, W1, b1, W2, b2, Wtc, btc):
    packed = pl.pallas_call(
        _pack_body,
        out_shape=jax.ShapeDtypeStruct((1, E), jnp.int32),
    )(edges).reshape(E)

    xT = pl.pallas_call(
        _in_proj_body,
        out_shape=jax.ShapeDtypeStruct((HIDDEN, N), jnp.float32),
    )(x, Wdr, bdr.reshape(HIDDEN, 1))

    mesh = plsc.VectorSubcoreMesh(core_axis_name="c", subcore_axis_name="s")
    cp = pltpu.CompilerParams(needs_layout_passes=False)
    sc_a = pl.kernel(
        _make_sc_body(True),
        out_type=[
            jax.ShapeDtypeStruct((N,), jnp.float32),
            jax.ShapeDtypeStruct((E,), jnp.float32),
            jax.ShapeDtypeStruct((HIDDEN, N), jnp.float32),
        ],
        mesh=mesh, compiler_params=cp, scratch_types=_SC_SCRATCH,
    )
    dinv_arr, norm_arr, x10T = sc_a(xT, packed)

    dcol = x10T.reshape(HIDDEN * N, 1)
    hcol = xT.reshape(HIDDEN * N, 1)
    xmT = pl.pallas_call(
        _mlp_body,
        grid=(HIDDEN * N // BR,),
        in_specs=[
            pl.BlockSpec((BR, 1), lambda i: (i, 0)),
            pl.BlockSpec((BR, 1), lambda i: (i, 0)),
            pl.BlockSpec((HIDDEN, EMB_DIM), lambda i: (0, 0)),
            pl.BlockSpec((2 + EMB_DIM, HID2), lambda i: (0, 0)),
            pl.BlockSpec((1, HID2), lambda i: (0, 0)),
            pl.BlockSpec((HID2, 1), lambda i: (0, 0)),
            pl.BlockSpec((1, 1), lambda i: (0, 0)),
        ],
        out_specs=pl.BlockSpec((BR, 1), lambda i: (i, 0)),
        out_shape=jax.ShapeDtypeStruct((HIDDEN * N, 1), jnp.float32),
    )(dcol, hcol, emb_table, W1, b1.reshape(1, HID2), W2,
      b2.reshape(1, 1)).reshape(HIDDEN, N)

    sc_b = pl.kernel(
        _make_sc_body(False),
        out_type=jax.ShapeDtypeStruct((HIDDEN, N), jnp.float32),
        mesh=mesh, compiler_params=cp, scratch_types=_SC_SCRATCH,
    )
    xT2 = sc_b(xmT, packed, dinv_arr, norm_arr)

    out = pl.pallas_call(
        _out_proj_body,
        out_shape=jax.ShapeDtypeStruct((N, CLASSES), jnp.float32),
    )(xT2, Wtc, btc.reshape(1, CLASSES))
    return out


# speedup vs baseline: 1.1010x; 1.0002x over previous
import jax
import jax.numpy as jnp
from jax import lax
from jax.experimental import pallas as pl
from jax.experimental.pallas import tpu as pltpu
from jax.experimental.pallas import tpu_sc as plsc

N = 10000
E = 320000
FEATS = 128
HIDDEN = 64
CLASSES = 64
DEPTH = 10
ALPHA = 0.9
EMB_DIM = 7
HID2 = 11

CHUNK = 10000
NCHUNK = E // CHUNK
NPAIR = NCHUNK // 2
EGRP = CHUNK // 16
NGRP = N // 16
MASK14 = (1 << 14) - 1
BR = 10000


def _pack_body(edges_ref, packed_ref):
    e = edges_ref[...]
    packed_ref[...] = e[0:1, :] + (e[1:2, :] << 14)


def _in_proj_body(x_ref, W_ref, b_ref, out_ref):
    yT = lax.dot_general(W_ref[...], x_ref[...], (((0,), (1,)), ((), ())),
                         preferred_element_type=jnp.float32)
    out_ref[...] = yT + b_ref[...]


def _out_proj_body(xT_ref, W_ref, b_ref, out_ref):
    y = lax.dot_general(xT_ref[...], W_ref[...], (((0,), (0,)), ((), ())),
                        preferred_element_type=jnp.float32)
    out_ref[...] = y + b_ref[...]


def _mlp_body(d_ref, h_ref, emb_ref, W1_ref, b1_ref, W2_ref, b2_ref,
              out_ref):
    d = d_ref[...]
    h = h_ref[...]
    e = jnp.broadcast_to(emb_ref[0:1, :], (BR, EMB_DIM))
    z = jnp.concatenate([d, h, e], axis=1)
    z = z @ W1_ref[...] + b1_ref[...]
    z = jnp.where(z >= 0, z, 0.01 * z)
    z = (z @ W2_ref[...] + b2_ref[...]) / 2.0
    out_ref[...] = z


def _make_sc_body(phase_a):
    def _sc_body(xT_hbm, packed_hbm, dinv_io_hbm, norm_io_hbm, out_hbm,
                 x0, x1, h0a, h0b, acc0, acc1, dinv, eb0, eb1, nb0, nb1,
                 sem0, sem1, sem2, sem3):
        wid = lax.axis_index("s") * 2 + lax.axis_index("c")
        c0 = wid * 2

        pltpu.sync_copy(xT_hbm.at[c0], x0)
        pltpu.sync_copy(xT_hbm.at[c0 + 1], x1)
        pltpu.sync_copy(xT_hbm.at[c0], h0a)
        pltpu.sync_copy(xT_hbm.at[c0 + 1], h0b)

        def stream_edges(group_fn, with_norm):
            pltpu.async_copy(packed_hbm.at[pl.ds(0, CHUNK)], eb0, sem0)
            pltpu.async_copy(packed_hbm.at[pl.ds(CHUNK, CHUNK)], eb1, sem1)
            if with_norm:
                pltpu.async_copy(norm_io_hbm.at[pl.ds(0, CHUNK)], nb0, sem2)
                pltpu.async_copy(norm_io_hbm.at[pl.ds(CHUNK, CHUNK)], nb1, sem3)

            def pair_body(p, carry):
                base = p * (2 * CHUNK)
                pltpu.make_async_copy(
                    packed_hbm.at[pl.ds(base, CHUNK)], eb0, sem0).wait()
                if with_norm:
                    pltpu.make_async_copy(
                        norm_io_hbm.at[pl.ds(base, CHUNK)], nb0, sem2).wait()

                    @plsc.parallel_loop(0, EGRP, unroll=2)
                    def _(g):
                        group_fn(eb0[pl.ds(g * 16, 16)],
                                 nb0[pl.ds(g * 16, 16)])
                else:
                    @plsc.parallel_loop(0, EGRP, unroll=2)
                    def _(g):
                        group_fn(eb0[pl.ds(g * 16, 16)], None)

                @pl.when(p + 1 < NPAIR)
                def _():
                    pltpu.async_copy(
                        packed_hbm.at[pl.ds(base + 2 * CHUNK, CHUNK)],
                        eb0, sem0)
                    if with_norm:
                        pltpu.async_copy(
                            norm_io_hbm.at[pl.ds(base + 2 * CHUNK, CHUNK)],
                            nb0, sem2)

                pltpu.make_async_copy(
                    packed_hbm.at[pl.ds(base + CHUNK, CHUNK)],
                    eb1, sem1).wait()
                if with_norm:
                    pltpu.make_async_copy(
                        norm_io_hbm.at[pl.ds(base + CHUNK, CHUNK)],
                        nb1, sem3).wait()

                    @plsc.parallel_loop(0, EGRP, unroll=2)
                    def _(g):
                        group_fn(eb1[pl.ds(g * 16, 16)],
                                 nb1[pl.ds(g * 16, 16)])
                else:
                    @plsc.parallel_loop(0, EGRP, unroll=2)
                    def _(g):
                        group_fn(eb1[pl.ds(g * 16, 16)], None)

                @pl.when(p + 1 < NPAIR)
                def _():
                    pltpu.async_copy(
                        packed_hbm.at[pl.ds(base + 3 * CHUNK, CHUNK)],
                        eb1, sem1)
                    if with_norm:
                        pltpu.async_copy(
                            norm_io_hbm.at[pl.ds(base + 3 * CHUNK, CHUNK)],
                            nb1, sem3)

                return carry

            lax.fori_loop(0, NPAIR, pair_body, 0)

        if phase_a:
            ones = jnp.full((16,), 1.0, jnp.float32)

            @plsc.parallel_loop(0, NGRP)
            def _(i):
                dinv[pl.ds(i * 16, 16)] = ones

            def deg_fn(pk, _nr):
                dstv = lax.shift_right_logical(pk, 14)
                plsc.addupdate_scatter(dinv, [dstv], ones)

            stream_edges(deg_fn, False)

            @plsc.parallel_loop(0, NGRP)
            def _(i):
                sl = pl.ds(i * 16, 16)
                v = dinv[sl]
                bits = lax.bitcast_convert_type(v, jnp.int32)
                y = lax.bitcast_convert_type(
                    jnp.int32(0x5F3759DF) - lax.shift_right_logical(bits, 1),
                    jnp.float32)
                for _unused in range(3):
                    y = y * (1.5 - 0.5 * v * y * y)
                dinv[sl] = y

            @pl.when(wid == 0)
            def _():
                pltpu.sync_copy(dinv, dinv_io_hbm)

            pltpu.async_copy(packed_hbm.at[pl.ds(0, CHUNK)], eb0, sem0)
            pltpu.async_copy(packed_hbm.at[pl.ds(CHUNK, CHUNK)], eb1, sem1)

            def _norm_groups(eb, nb):
                @plsc.parallel_loop(0, EGRP, unroll=2)
                def _(g):
                    sl = pl.ds(g * 16, 16)
                    pk = eb[sl]
                    srcv = pk & MASK14
                    dstv = lax.shift_right_logical(pk, 14)
                    nb[sl] = (plsc.load_gather(dinv, [srcv])
                              * plsc.load_gather(dinv, [dstv]))

            def npair_body(pr, carry):
                base = pr * (2 * CHUNK)
                pltpu.make_async_copy(
                    packed_hbm.at[pl.ds(base, CHUNK)], eb0, sem0).wait()
                @pl.when(pr > 0)
                def _():
                    pltpu.make_async_copy(
                        nb0, norm_io_hbm.at[pl.ds(base - 2 * CHUNK, CHUNK)],
                        sem2).wait()
                _norm_groups(eb0, nb0)
                pltpu.async_copy(
                    nb0, norm_io_hbm.at[pl.ds(base, CHUNK)], sem2)
                @pl.when(pr + 1 < NPAIR)
                def _():
                    pltpu.async_copy(
                        packed_hbm.at[pl.ds(base + 2 * CHUNK, CHUNK)],
                        eb0, sem0)

                pltpu.make_async_copy(
                    packed_hbm.at[pl.ds(base + CHUNK, CHUNK)], eb1, sem1).wait()
                @pl.when(pr > 0)
                def _():
                    pltpu.make_async_copy(
                        nb1, norm_io_hbm.at[pl.ds(base - CHUNK, CHUNK)],
                        sem3).wait()
                _norm_groups(eb1, nb1)
                pltpu.async_copy(
                    nb1, norm_io_hbm.at[pl.ds(base + CHUNK, CHUNK)], sem3)
                @pl.when(pr + 1 < NPAIR)
                def _():
                    pltpu.async_copy(
                        packed_hbm.at[pl.ds(base + 3 * CHUNK, CHUNK)],
                        eb1, sem1)
                return carry

            lax.fori_loop(0, NPAIR, npair_body, 0)
            pltpu.make_async_copy(
                nb0, norm_io_hbm.at[pl.ds((NCHUNK - 2) * CHUNK, CHUNK)],
                sem2).wait()
            pltpu.make_async_copy(
                nb1, norm_io_hbm.at[pl.ds((NCHUNK - 1) * CHUNK, CHUNK)],
                sem3).wait()
        else:
            pltpu.sync_copy(dinv_io_hbm, dinv)

        def conv_fn(pk, nrm):
            srcv = pk & MASK14
            dstv = lax.shift_right_logical(pk, 14)
            plsc.addupdate_scatter(acc0, [dstv],
                                   plsc.load_gather(x0, [srcv]) * nrm)
            plsc.addupdate_scatter(acc1, [dstv],
                                   plsc.load_gather(x1, [srcv]) * nrm)

        def it_body(it, carry):
            @plsc.parallel_loop(0, NGRP)
            def _(i):
                sl = pl.ds(i * 16, 16)
                d2 = dinv[sl]
                d2 = d2 * d2
                acc0[sl] = d2 * x0[sl]
                acc1[sl] = d2 * x1[sl]

            stream_edges(conv_fn, True)

            @plsc.parallel_loop(0, NGRP)
            def _(i):
                sl = pl.ds(i * 16, 16)
                x0[sl] = ALPHA * acc0[sl] + (1.0 - ALPHA) * h0a[sl]
                x1[sl] = ALPHA * acc1[sl] + (1.0 - ALPHA) * h0b[sl]

            return carry

        lax.fori_loop(0, DEPTH, it_body, 0)

        pltpu.sync_copy(x0, out_hbm.at[c0])
        pltpu.sync_copy(x1, out_hbm.at[c0 + 1])

    return _sc_body


_SC_SCRATCH = [
    pltpu.VMEM((N,), jnp.float32),
    pltpu.VMEM((N,), jnp.float32),
    pltpu.VMEM((N,), jnp.float32),
    pltpu.VMEM((N,), jnp.float32),
    pltpu.VMEM((N,), jnp.float32),
    pltpu.VMEM((N,), jnp.float32),
    pltpu.VMEM((N,), jnp.float32),
    pltpu.VMEM((CHUNK,), jnp.int32),
    pltpu.VMEM((CHUNK,), jnp.int32),
    pltpu.VMEM((CHUNK,), jnp.float32),
    pltpu.VMEM((CHUNK,), jnp.float32),
    pltpu.SemaphoreType.DMA,
    pltpu.SemaphoreType.DMA,
    pltpu.SemaphoreType.DMA,
    pltpu.SemaphoreType.DMA,
]


def kernel(x, edges, Wdr, bdr, emb_table, W1, b1, W2, b2, Wtc, btc):
    packed = pl.pallas_call(
        _pack_body,
        out_shape=jax.ShapeDtypeStruct((1, E), jnp.int32),
    )(edges).reshape(E)

    xT = pl.pallas_call(
        _in_proj_body,
        out_shape=jax.ShapeDtypeStruct((HIDDEN, N), jnp.float32),
    )(x, Wdr, bdr.reshape(HIDDEN, 1))

    mesh = plsc.VectorSubcoreMesh(core_axis_name="c", subcore_axis_name="s")
    cp = pltpu.CompilerParams(needs_layout_passes=False)
    sc_a = pl.kernel(
        _make_sc_body(True),
        out_type=[
            jax.ShapeDtypeStruct((N,), jnp.float32),
            jax.ShapeDtypeStruct((E,), jnp.float32),
            jax.ShapeDtypeStruct((HIDDEN, N), jnp.float32),
        ],
        mesh=mesh, compiler_params=cp, scratch_types=_SC_SCRATCH,
    )
    dinv_arr, norm_arr, x10T = sc_a(xT, packed)

    dcol = x10T.reshape(HIDDEN * N, 1)
    hcol = xT.reshape(HIDDEN * N, 1)
    xmT = pl.pallas_call(
        _mlp_body,
        grid=(HIDDEN * N // BR,),
        in_specs=[
            pl.BlockSpec((BR, 1), lambda i: (i, 0)),
            pl.BlockSpec((BR, 1), lambda i: (i, 0)),
            pl.BlockSpec((HIDDEN, EMB_DIM), lambda i: (0, 0)),
            pl.BlockSpec((2 + EMB_DIM, HID2), lambda i: (0, 0)),
            pl.BlockSpec((1, HID2), lambda i: (0, 0)),
            pl.BlockSpec((HID2, 1), lambda i: (0, 0)),
            pl.BlockSpec((1, 1), lambda i: (0, 0)),
        ],
        out_specs=pl.BlockSpec((BR, 1), lambda i: (i, 0)),
        out_shape=jax.ShapeDtypeStruct((HIDDEN * N, 1), jnp.float32),
    )(dcol, hcol, emb_table, W1, b1.reshape(1, HID2), W2,
      b2.reshape(1, 1)).reshape(HIDDEN, N)

    sc_b = pl.kernel(
        _make_sc_body(False),
        out_type=jax.ShapeDtypeStruct((HIDDEN, N), jnp.float32),
        mesh=mesh, compiler_params=cp, scratch_types=_SC_SCRATCH,
    )
    xT2 = sc_b(xmT, packed, dinv_arr, norm_arr)

    out = pl.pallas_call(
        _out_proj_body,
        out_shape=jax.ShapeDtypeStruct((N, CLASSES), jnp.float32),
    )(xT2, Wtc, btc.reshape(1, CLASSES))
    return out
